# Initial kernel scaffold; baseline (speedup 1.0000x reference)
import jax
import jax.numpy as jnp
from jax.experimental import pallas as pl


def kernel(x, Wq, bq, Wk, bk, Wv, bv, Wo, bo):
    raise NotImplementedError("write your pallas kernel here")


# speedup vs baseline: 17.7386x; 17.7386x over previous
import jax
import jax.numpy as jnp
from jax.experimental import pallas as pl

S = 2048
D = 768
H = 12
DH = D // H
BQ = 256
NQ = S // BQ

_DN_T = (((1,), (1,)), ((), ()))
_DN_N = (((1,), (0,)), ((), ()))


def _qkv_body(x_ref, wq_ref, bq_ref, wk_ref, bk_ref, wv_ref, bv_ref,
              q_ref, k_ref, v_ref):
    xb = x_ref[...]
    qb = jax.lax.dot_general(
        xb, wq_ref[...], _DN_T, preferred_element_type=jnp.float32) + bq_ref[...]
    kb = jax.lax.dot_general(
        xb, wk_ref[...], _DN_T, preferred_element_type=jnp.float32) + bk_ref[...]
    vb = jax.lax.dot_general(
        xb, wv_ref[...], _DN_T, preferred_element_type=jnp.float32) + bv_ref[...]
    for h in range(H):
        sl = slice(h * DH, (h + 1) * DH)
        q_ref[h, :, :] = qb[:, sl]
        k_ref[h, :, :] = kb[:, sl]
        v_ref[h, :, :] = vb[:, sl]


def _attn_body(q_ref, k_ref, v_ref, wot_ref, bo_ref, out_ref):
    h = pl.program_id(1)
    s = jax.lax.dot_general(q_ref[0], k_ref[0], _DN_T,
                            preferred_element_type=jnp.float32)
    m = jnp.max(s, axis=1, keepdims=True)
    colid = jax.lax.broadcasted_iota(jnp.int32, (BQ, S), 1)
    idx = jnp.min(jnp.where(s == m, colid, S), axis=1, keepdims=True)
    onehot = (colid == idx).astype(jnp.float32)
    att = jax.lax.dot_general(onehot, v_ref[0], _DN_N,
                              preferred_element_type=jnp.float32,
                              precision=jax.lax.Precision.HIGHEST)
    proj = jax.lax.dot_general(att, wot_ref[...], _DN_N,
                               preferred_element_type=jnp.float32,
                               precision=jax.lax.Precision.HIGHEST)

    @pl.when(h == 0)
    def _():
        out_ref[...] = proj + bo_ref[...]

    @pl.when(h != 0)
    def _():
        out_ref[...] += proj


def kernel(x, Wq, bq, Wk, bk, Wv, bv, Wo, bo):
    x2 = x.reshape(S, D)
    bq2 = bq.reshape(1, D)
    bk2 = bk.reshape(1, D)
    bv2 = bv.reshape(1, D)
    bo2 = bo.reshape(1, D)
    WoT = Wo.T

    w_spec = pl.BlockSpec((D, D), lambda i: (0, 0))
    b_spec = pl.BlockSpec((1, D), lambda i: (0, 0))
    hm_spec = pl.BlockSpec((H, BQ, DH), lambda i: (0, i, 0))
    q, k, v = pl.pallas_call(
        _qkv_body,
        grid=(NQ,),
        in_specs=[pl.BlockSpec((BQ, D), lambda i: (i, 0)),
                  w_spec, b_spec, w_spec, b_spec, w_spec, b_spec],
        out_specs=[hm_spec, hm_spec, hm_spec],
        out_shape=[jax.ShapeDtypeStruct((H, S, DH), jnp.float32)] * 3,
    )(x2, Wq, bq2, Wk, bk2, Wv, bv2)

    out = pl.pallas_call(
        _attn_body,
        grid=(NQ, H),
        in_specs=[
            pl.BlockSpec((1, BQ, DH), lambda i, h: (h, i, 0)),
            pl.BlockSpec((1, S, DH), lambda i, h: (h, 0, 0)),
            pl.BlockSpec((1, S, DH), lambda i, h: (h, 0, 0)),
            pl.BlockSpec((DH, D), lambda i, h: (h, 0)),
            pl.BlockSpec((1, D), lambda i, h: (0, 0)),
        ],
        out_specs=pl.BlockSpec((BQ, D), lambda i, h: (i, 0)),
        out_shape=jax.ShapeDtypeStruct((S, D), jnp.float32),
    )(q, k, v, WoT, bo2)

    return out.reshape(1, S, D)


# speedup vs baseline: 40.9977x; 2.3112x over previous
import jax
import jax.numpy as jnp
from jax.experimental import pallas as pl

S = 2048
D = 768
H = 12
DH = D // H
BQ = 256
NQ = S // BQ

_DN_T = (((1,), (1,)), ((), ()))
_DN_N = (((1,), (0,)), ((), ()))


def _qkv_body(x_ref, wq_ref, bq_ref, wk_ref, bk_ref, wv_ref, bv_ref,
              q_ref, k_ref, v_ref):
    xb = x_ref[...]
    qb = jax.lax.dot_general(
        xb, wq_ref[...], _DN_T, preferred_element_type=jnp.float32) + bq_ref[...]
    kb = jax.lax.dot_general(
        xb, wk_ref[...], _DN_T, preferred_element_type=jnp.float32) + bk_ref[...]
    vb = jax.lax.dot_general(
        xb, wv_ref[...], _DN_T, preferred_element_type=jnp.float32) + bv_ref[...]
    for h in range(H):
        sl = slice(h * DH, (h + 1) * DH)
        q_ref[h, :, :] = qb[:, sl]
        k_ref[h, :, :] = kb[:, sl]
        v_ref[h, :, :] = vb[:, sl]


def _attn_body(q_ref, k_ref, v_ref, wot_ref, bo_ref, out_ref):
    h = pl.program_id(1)
    s = jax.lax.dot_general(q_ref[0], k_ref[0], _DN_T,
                            preferred_element_type=jnp.float32)
    m = jnp.max(s, axis=1, keepdims=True)
    rev = jax.lax.broadcasted_iota(jnp.int32, (BQ, S), 1)
    t = jnp.where(s == m, S - rev, 0)
    mx = jnp.max(t, axis=1, keepdims=True)
    onehot = (t == mx).astype(jnp.float32)
    att = jax.lax.dot_general(onehot, v_ref[0], _DN_N,
                              preferred_element_type=jnp.float32)
    proj = jax.lax.dot_general(att, wot_ref[...], _DN_N,
                               preferred_element_type=jnp.float32)

    @pl.when(h == 0)
    def _():
        out_ref[...] = proj + bo_ref[...]

    @pl.when(h != 0)
    def _():
        out_ref[...] += proj


def kernel(x, Wq, bq, Wk, bk, Wv, bv, Wo, bo):
    x2 = x.reshape(S, D)
    bq2 = bq.reshape(1, D)
    bk2 = bk.reshape(1, D)
    bv2 = bv.reshape(1, D)
    bo2 = bo.reshape(1, D)
    WoT = Wo.T

    w_spec = pl.BlockSpec((D, D), lambda i: (0, 0))
    b_spec = pl.BlockSpec((1, D), lambda i: (0, 0))
    hm_spec = pl.BlockSpec((H, BQ, DH), lambda i: (0, i, 0))
    q, k, v = pl.pallas_call(
        _qkv_body,
        grid=(NQ,),
        in_specs=[pl.BlockSpec((BQ, D), lambda i: (i, 0)),
                  w_spec, b_spec, w_spec, b_spec, w_spec, b_spec],
        out_specs=[hm_spec, hm_spec, hm_spec],
        out_shape=[jax.ShapeDtypeStruct((H, S, DH), jnp.float32)] * 3,
    )(x2, Wq, bq2, Wk, bk2, Wv, bv2)

    out = pl.pallas_call(
        _attn_body,
        grid=(NQ, H),
        in_specs=[
            pl.BlockSpec((1, BQ, DH), lambda i, h: (h, i, 0)),
            pl.BlockSpec((1, S, DH), lambda i, h: (h, 0, 0)),
            pl.BlockSpec((1, S, DH), lambda i, h: (h, 0, 0)),
            pl.BlockSpec((DH, D), lambda i, h: (h, 0)),
            pl.BlockSpec((1, D), lambda i, h: (0, 0)),
        ],
        out_specs=pl.BlockSpec((BQ, D), lambda i, h: (i, 0)),
        out_shape=jax.ShapeDtypeStruct((S, D), jnp.float32),
    )(q, k, v, WoT, bo2)

    return out.reshape(1, S, D)


# speedup vs baseline: 50.9840x; 1.2436x over previous
import jax
import jax.numpy as jnp
from jax.experimental import pallas as pl

S = 2048
D = 768
H = 12
DH = D // H
BQ = 512
NQ = S // BQ

_DN_T = (((1,), (1,)), ((), ()))
_DN_N = (((1,), (0,)), ((), ()))


def _qkv_body(x_ref, wq_ref, bq_ref, wk_ref, bk_ref, wv_ref, bv_ref,
              q_ref, k_ref, v_ref):
    xb = x_ref[...]
    qb = jax.lax.dot_general(
        xb, wq_ref[...], _DN_T, preferred_element_type=jnp.float32) + bq_ref[...]
    kb = jax.lax.dot_general(
        xb, wk_ref[...], _DN_T, preferred_element_type=jnp.float32) + bk_ref[...]
    vb = jax.lax.dot_general(
        xb, wv_ref[...], _DN_T, preferred_element_type=jnp.float32) + bv_ref[...]
    for h in range(H):
        sl = slice(h * DH, (h + 1) * DH)
        q_ref[h, :, :] = qb[:, sl]
        k_ref[h, :, :] = kb[:, sl]
        v_ref[h, :, :] = vb[:, sl]


def _attn_body(q_ref, k_ref, v_ref, wot_ref, bo_ref, out_ref):
    h = pl.program_id(1)
    s = jax.lax.dot_general(q_ref[0], k_ref[0], _DN_T,
                            preferred_element_type=jnp.float32)
    m = jnp.max(s, axis=1, keepdims=True)
    rev = (S - jax.lax.broadcasted_iota(jnp.int32, (BQ, S), 1)).astype(jnp.float32)
    t = jnp.where(s == m, rev, 0.0)
    mx = jnp.max(t, axis=1, keepdims=True)
    onehot = (t == mx).astype(jnp.float32)
    att = jax.lax.dot_general(onehot, v_ref[0], _DN_N,
                              preferred_element_type=jnp.float32)
    proj = jax.lax.dot_general(att, wot_ref[...], _DN_N,
                               preferred_element_type=jnp.float32)

    @pl.when(h == 0)
    def _():
        out_ref[...] = proj + bo_ref[...]

    @pl.when(h != 0)
    def _():
        out_ref[...] += proj


def kernel(x, Wq, bq, Wk, bk, Wv, bv, Wo, bo):
    x2 = x.reshape(S, D)
    bq2 = bq.reshape(1, D)
    bk2 = bk.reshape(1, D)
    bv2 = bv.reshape(1, D)
    bo2 = bo.reshape(1, D)
    WoT = Wo.T

    w_spec = pl.BlockSpec((D, D), lambda i: (0, 0))
    b_spec = pl.BlockSpec((1, D), lambda i: (0, 0))
    hm_spec = pl.BlockSpec((H, BQ, DH), lambda i: (0, i, 0))
    q, k, v = pl.pallas_call(
        _qkv_body,
        grid=(NQ,),
        in_specs=[pl.BlockSpec((BQ, D), lambda i: (i, 0)),
                  w_spec, b_spec, w_spec, b_spec, w_spec, b_spec],
        out_specs=[hm_spec, hm_spec, hm_spec],
        out_shape=[jax.ShapeDtypeStruct((H, S, DH), jnp.float32)] * 3,
    )(x2, Wq, bq2, Wk, bk2, Wv, bv2)

    out = pl.pallas_call(
        _attn_body,
        grid=(NQ, H),
        in_specs=[
            pl.BlockSpec((1, BQ, DH), lambda i, h: (h, i, 0)),
            pl.BlockSpec((1, S, DH), lambda i, h: (h, 0, 0)),
            pl.BlockSpec((1, S, DH), lambda i, h: (h, 0, 0)),
            pl.BlockSpec((DH, D), lambda i, h: (h, 0)),
            pl.BlockSpec((1, D), lambda i, h: (0, 0)),
        ],
        out_specs=pl.BlockSpec((BQ, D), lambda i, h: (i, 0)),
        out_shape=jax.ShapeDtypeStruct((S, D), jnp.float32),
    )(q, k, v, WoT, bo2)

    return out.reshape(1, S, D)


# speedup vs baseline: 54.4042x; 1.0671x over previous
import jax
import jax.numpy as jnp
from jax.experimental import pallas as pl

S = 2048
D = 768
H = 12
DH = D // H
BQ = 1024
NQ = S // BQ

_DN_T = (((1,), (1,)), ((), ()))
_DN_N = (((1,), (0,)), ((), ()))


def _qkv_body(x_ref, wq_ref, bq_ref, wk_ref, bk_ref, wv_ref, bv_ref,
              q_ref, k_ref, v_ref):
    xb = x_ref[...]
    qb = jax.lax.dot_general(
        xb, wq_ref[...], _DN_T, preferred_element_type=jnp.float32) + bq_ref[...]
    kb = jax.lax.dot_general(
        xb, wk_ref[...], _DN_T, preferred_element_type=jnp.float32) + bk_ref[...]
    vb = jax.lax.dot_general(
        xb, wv_ref[...], _DN_T, preferred_element_type=jnp.float32) + bv_ref[...]
    for h in range(H):
        sl = slice(h * DH, (h + 1) * DH)
        q_ref[h, :, :] = qb[:, sl]
        k_ref[h, :, :] = kb[:, sl]
        v_ref[h, :, :] = vb[:, sl]


def _attn_body(q_ref, k_ref, v_ref, wot_ref, bo_ref, out_ref):
    h = pl.program_id(1)
    s = jax.lax.dot_general(q_ref[0], k_ref[0], _DN_T,
                            preferred_element_type=jnp.float32)
    m = jnp.max(s, axis=1, keepdims=True)
    rev = (S - jax.lax.broadcasted_iota(jnp.int32, (BQ, S), 1)).astype(jnp.float32)
    t = jnp.where(s == m, rev, 0.0)
    mx = jnp.max(t, axis=1, keepdims=True)
    onehot = (t == mx).astype(jnp.float32)
    att = jax.lax.dot_general(onehot, v_ref[0], _DN_N,
                              preferred_element_type=jnp.float32)
    proj = jax.lax.dot_general(att, wot_ref[...], _DN_N,
                               preferred_element_type=jnp.float32)

    @pl.when(h == 0)
    def _():
        out_ref[...] = proj + bo_ref[...]

    @pl.when(h != 0)
    def _():
        out_ref[...] += proj


def kernel(x, Wq, bq, Wk, bk, Wv, bv, Wo, bo):
    x2 = x.reshape(S, D)
    bq2 = bq.reshape(1, D)
    bk2 = bk.reshape(1, D)
    bv2 = bv.reshape(1, D)
    bo2 = bo.reshape(1, D)
    WoT = Wo.T

    w_spec = pl.BlockSpec((D, D), lambda i: (0, 0))
    b_spec = pl.BlockSpec((1, D), lambda i: (0, 0))
    hm_spec = pl.BlockSpec((H, BQ, DH), lambda i: (0, i, 0))
    q, k, v = pl.pallas_call(
        _qkv_body,
        grid=(NQ,),
        in_specs=[pl.BlockSpec((BQ, D), lambda i: (i, 0)),
                  w_spec, b_spec, w_spec, b_spec, w_spec, b_spec],
        out_specs=[hm_spec, hm_spec, hm_spec],
        out_shape=[jax.ShapeDtypeStruct((H, S, DH), jnp.float32)] * 3,
    )(x2, Wq, bq2, Wk, bk2, Wv, bv2)

    out = pl.pallas_call(
        _attn_body,
        grid=(NQ, H),
        in_specs=[
            pl.BlockSpec((1, BQ, DH), lambda i, h: (h, i, 0)),
            pl.BlockSpec((1, S, DH), lambda i, h: (h, 0, 0)),
            pl.BlockSpec((1, S, DH), lambda i, h: (h, 0, 0)),
            pl.BlockSpec((DH, D), lambda i, h: (h, 0)),
            pl.BlockSpec((1, D), lambda i, h: (0, 0)),
        ],
        out_specs=pl.BlockSpec((BQ, D), lambda i, h: (i, 0)),
        out_shape=jax.ShapeDtypeStruct((S, D), jnp.float32),
    )(q, k, v, WoT, bo2)

    return out.reshape(1, S, D)


# speedup vs baseline: 56.8351x; 1.0447x over previous
import jax
import jax.numpy as jnp
from jax.experimental import pallas as pl

S = 2048
D = 768
H = 12
DH = D // H
BQ = 2048
NQ = S // BQ

_DN_T = (((1,), (1,)), ((), ()))
_DN_N = (((1,), (0,)), ((), ()))


def _qkv_body(x_ref, wq_ref, bq_ref, wk_ref, bk_ref, wv_ref, bv_ref,
              q_ref, k_ref, v_ref):
    xb = x_ref[...]
    qb = jax.lax.dot_general(
        xb, wq_ref[...], _DN_T, preferred_element_type=jnp.float32) + bq_ref[...]
    kb = jax.lax.dot_general(
        xb, wk_ref[...], _DN_T, preferred_element_type=jnp.float32) + bk_ref[...]
    vb = jax.lax.dot_general(
        xb, wv_ref[...], _DN_T, preferred_element_type=jnp.float32) + bv_ref[...]
    for h in range(H):
        sl = slice(h * DH, (h + 1) * DH)
        q_ref[h, :, :] = qb[:, sl]
        k_ref[h, :, :] = kb[:, sl]
        v_ref[h, :, :] = vb[:, sl]


def _attn_body(q_ref, k_ref, v_ref, wot_ref, bo_ref, out_ref):
    h = pl.program_id(1)
    s = jax.lax.dot_general(q_ref[0], k_ref[0], _DN_T,
                            preferred_element_type=jnp.float32)
    m = jnp.max(s, axis=1, keepdims=True)
    rev = (S - jax.lax.broadcasted_iota(jnp.int32, (BQ, S), 1)).astype(jnp.float32)
    t = jnp.where(s == m, rev, 0.0)
    mx = jnp.max(t, axis=1, keepdims=True)
    onehot = (t == mx).astype(jnp.float32)
    att = jax.lax.dot_general(onehot, v_ref[0], _DN_N,
                              preferred_element_type=jnp.float32)
    proj = jax.lax.dot_general(att, wot_ref[...], _DN_N,
                               preferred_element_type=jnp.float32)

    @pl.when(h == 0)
    def _():
        out_ref[...] = proj + bo_ref[...]

    @pl.when(h != 0)
    def _():
        out_ref[...] += proj


def kernel(x, Wq, bq, Wk, bk, Wv, bv, Wo, bo):
    x2 = x.reshape(S, D)
    bq2 = bq.reshape(1, D)
    bk2 = bk.reshape(1, D)
    bv2 = bv.reshape(1, D)
    bo2 = bo.reshape(1, D)
    WoT = Wo.T

    w_spec = pl.BlockSpec((D, D), lambda i: (0, 0))
    b_spec = pl.BlockSpec((1, D), lambda i: (0, 0))
    hm_spec = pl.BlockSpec((H, BQ, DH), lambda i: (0, i, 0))
    q, k, v = pl.pallas_call(
        _qkv_body,
        grid=(NQ,),
        in_specs=[pl.BlockSpec((BQ, D), lambda i: (i, 0)),
                  w_spec, b_spec, w_spec, b_spec, w_spec, b_spec],
        out_specs=[hm_spec, hm_spec, hm_spec],
        out_shape=[jax.ShapeDtypeStruct((H, S, DH), jnp.float32)] * 3,
    )(x2, Wq, bq2, Wk, bk2, Wv, bv2)

    out = pl.pallas_call(
        _attn_body,
        grid=(NQ, H),
        in_specs=[
            pl.BlockSpec((1, BQ, DH), lambda i, h: (h, i, 0)),
            pl.BlockSpec((1, S, DH), lambda i, h: (h, 0, 0)),
            pl.BlockSpec((1, S, DH), lambda i, h: (h, 0, 0)),
            pl.BlockSpec((DH, D), lambda i, h: (h, 0)),
            pl.BlockSpec((1, D), lambda i, h: (0, 0)),
        ],
        out_specs=pl.BlockSpec((BQ, D), lambda i, h: (i, 0)),
        out_shape=jax.ShapeDtypeStruct((S, D), jnp.float32),
    )(q, k, v, WoT, bo2)

    return out.reshape(1, S, D)


# speedup vs baseline: 59.9130x; 1.0542x over previous
import jax
import jax.numpy as jnp
from jax.experimental import pallas as pl

S = 2048
D = 768
H = 12
DH = D // H
BQ = 2048
NQ = S // BQ

_DN_T = (((1,), (1,)), ((), ()))
_DN_N = (((1,), (0,)), ((), ()))


def _qkv_body(x_ref, wq_ref, bq_ref, wk_ref, bk_ref, wv_ref, bv_ref,
              q_ref, k_ref, v_ref):
    xb = x_ref[...]
    qb = jax.lax.dot_general(
        xb, wq_ref[...], _DN_T, preferred_element_type=jnp.float32) + bq_ref[...]
    kb = jax.lax.dot_general(
        xb, wk_ref[...], _DN_T, preferred_element_type=jnp.float32) + bk_ref[...]
    vb = jax.lax.dot_general(
        xb, wv_ref[...], _DN_T, preferred_element_type=jnp.float32) + bv_ref[...]
    for h in range(H):
        sl = slice(h * DH, (h + 1) * DH)
        q_ref[h, :, :] = qb[:, sl]
        k_ref[h, :, :] = kb[:, sl]
        v_ref[h, :, :] = vb[:, sl]


def _attn_body(q_ref, k_ref, v_ref, wot_ref, bo_ref, out_ref):
    h = pl.program_id(1)
    s = jax.lax.dot_general(q_ref[0], k_ref[0], _DN_T,
                            preferred_element_type=jnp.float32)
    idx = jnp.argmax(s, axis=1).astype(jnp.int32)
    colid = jax.lax.broadcasted_iota(jnp.int32, (BQ, S), 1)
    onehot = (colid == idx[:, None]).astype(jnp.float32)
    att = jax.lax.dot_general(onehot, v_ref[0], _DN_N,
                              preferred_element_type=jnp.float32)
    proj = jax.lax.dot_general(att, wot_ref[...], _DN_N,
                               preferred_element_type=jnp.float32)

    @pl.when(h == 0)
    def _():
        out_ref[...] = proj + bo_ref[...]

    @pl.when(h != 0)
    def _():
        out_ref[...] += proj


def kernel(x, Wq, bq, Wk, bk, Wv, bv, Wo, bo):
    x2 = x.reshape(S, D)
    bq2 = bq.reshape(1, D)
    bk2 = bk.reshape(1, D)
    bv2 = bv.reshape(1, D)
    bo2 = bo.reshape(1, D)
    WoT = Wo.T

    w_spec = pl.BlockSpec((D, D), lambda i: (0, 0))
    b_spec = pl.BlockSpec((1, D), lambda i: (0, 0))
    hm_spec = pl.BlockSpec((H, BQ, DH), lambda i: (0, i, 0))
    q, k, v = pl.pallas_call(
        _qkv_body,
        grid=(NQ,),
        in_specs=[pl.BlockSpec((BQ, D), lambda i: (i, 0)),
                  w_spec, b_spec, w_spec, b_spec, w_spec, b_spec],
        out_specs=[hm_spec, hm_spec, hm_spec],
        out_shape=[jax.ShapeDtypeStruct((H, S, DH), jnp.float32)] * 3,
    )(x2, Wq, bq2, Wk, bk2, Wv, bv2)

    out = pl.pallas_call(
        _attn_body,
        grid=(NQ, H),
        in_specs=[
            pl.BlockSpec((1, BQ, DH), lambda i, h: (h, i, 0)),
            pl.BlockSpec((1, S, DH), lambda i, h: (h, 0, 0)),
            pl.BlockSpec((1, S, DH), lambda i, h: (h, 0, 0)),
            pl.BlockSpec((DH, D), lambda i, h: (h, 0)),
            pl.BlockSpec((1, D), lambda i, h: (0, 0)),
        ],
        out_specs=pl.BlockSpec((BQ, D), lambda i, h: (i, 0)),
        out_shape=jax.ShapeDtypeStruct((S, D), jnp.float32),
    )(q, k, v, WoT, bo2)

    return out.reshape(1, S, D)


# speedup vs baseline: 71.4050x; 1.1918x over previous
import functools

import jax
import jax.numpy as jnp
from jax import lax
from jax.experimental import pallas as pl
from jax.experimental.pallas import tpu as pltpu
from jax.experimental.pallas import tpu_sc as plsc

S = 2048
D = 768
H = 12
DH = D // H
NW = 32
RW = H * S // NW
NCH = RW // 128

_DN_T = (((1,), (1,)), ((), ()))
_DN_N = (((1,), (0,)), ((), ()))


def _qkv_body(x_ref, wq_ref, bq_ref, wk_ref, bk_ref, wv_ref, bv_ref,
              q_ref, k_ref, v_ref):
    xb = x_ref[...]
    qb = jax.lax.dot_general(
        xb, wq_ref[...], _DN_T, preferred_element_type=jnp.float32) + bq_ref[...]
    kb = jax.lax.dot_general(
        xb, wk_ref[...], _DN_T, preferred_element_type=jnp.float32) + bk_ref[...]
    vb = jax.lax.dot_general(
        xb, wv_ref[...], _DN_T, preferred_element_type=jnp.float32) + bv_ref[...]
    for h in range(H):
        sl = slice(h * DH, (h + 1) * DH)
        q_ref[h, :, :] = qb[:, sl]
        k_ref[h, :, :] = kb[:, sl]
        v_ref[h, :, 0:DH] = vb[:, sl]
        v_ref[h, :, DH:2 * DH] = vb[:, sl]


def _score_idx_body(q_ref, k_ref, idx_ref):
    h = pl.program_id(0)
    s = jax.lax.dot_general(q_ref[0], k_ref[0], _DN_T,
                            preferred_element_type=jnp.float32)
    idx = jnp.argmax(s, axis=1).astype(jnp.int32) + h * S
    idx_ref[0] = idx[:, None]


def _make_gather():
    mesh = plsc.VectorSubcoreMesh(core_axis_name="c", subcore_axis_name="s")

    @functools.partial(
        pl.kernel, mesh=mesh,
        out_type=jax.ShapeDtypeStruct((H * S, 2 * DH), jnp.float32),
        scratch_types=[
            pltpu.VMEM((NCH, 128), jnp.int32),
            pltpu.VMEM((RW, 2 * DH), jnp.float32),
            pltpu.SemaphoreType.DMA,
        ],
    )
    def gather(table_hbm, idx_hbm, out_hbm, idx_v, rows_v, sem):
        wid = lax.axis_index("s") * 2 + lax.axis_index("c")
        pltpu.sync_copy(idx_hbm.at[wid], idx_v)
        for j in range(NCH):
            pltpu.async_copy(table_hbm.at[idx_v.at[j]],
                             rows_v.at[pl.ds(j * 128, 128)], sem).wait()
        pltpu.sync_copy(rows_v, out_hbm.at[pl.ds(wid * RW, RW)])

    return gather


def _proj_body(att_ref, wot_ref, bo_ref, out_ref):
    h = pl.program_id(0)
    proj = jax.lax.dot_general(att_ref[0, :, 0:DH], wot_ref[...], _DN_N,
                               preferred_element_type=jnp.float32)

    @pl.when(h == 0)
    def _():
        out_ref[...] = proj + bo_ref[...]

    @pl.when(h != 0)
    def _():
        out_ref[...] += proj


def kernel(x, Wq, bq, Wk, bk, Wv, bv, Wo, bo):
    x2 = x.reshape(S, D)
    bq2 = bq.reshape(1, D)
    bk2 = bk.reshape(1, D)
    bv2 = bv.reshape(1, D)
    bo2 = bo.reshape(1, D)
    WoT = Wo.T

    w_spec = pl.BlockSpec((D, D), lambda: (0, 0))
    b_spec = pl.BlockSpec((1, D), lambda: (0, 0))
    hm_spec = pl.BlockSpec((H, S, DH), lambda: (0, 0, 0))
    vp_spec = pl.BlockSpec((H, S, 2 * DH), lambda: (0, 0, 0))
    q, k, v = pl.pallas_call(
        _qkv_body,
        grid=(),
        in_specs=[pl.BlockSpec((S, D), lambda: (0, 0)),
                  w_spec, b_spec, w_spec, b_spec, w_spec, b_spec],
        out_specs=[hm_spec, hm_spec, vp_spec],
        out_shape=[jax.ShapeDtypeStruct((H, S, DH), jnp.float32)] * 2
        + [jax.ShapeDtypeStruct((H, S, 2 * DH), jnp.float32)],
    )(x2, Wq, bq2, Wk, bk2, Wv, bv2)

    idx = pl.pallas_call(
        _score_idx_body,
        grid=(H,),
        in_specs=[
            pl.BlockSpec((1, S, DH), lambda h: (h, 0, 0)),
            pl.BlockSpec((1, S, DH), lambda h: (h, 0, 0)),
        ],
        out_specs=pl.BlockSpec((1, S, 1), lambda h: (h, 0, 0)),
        out_shape=jax.ShapeDtypeStruct((H, S, 1), jnp.int32),
    )(q, k)

    table = v.reshape(H * S, 2 * DH)
    idx_w = idx.reshape(NW, NCH, 128)
    att = _make_gather()(table, idx_w)

    out = pl.pallas_call(
        _proj_body,
        grid=(H,),
        in_specs=[
            pl.BlockSpec((1, S, 2 * DH), lambda h: (h, 0, 0)),
            pl.BlockSpec((DH, D), lambda h: (h, 0)),
            pl.BlockSpec((1, D), lambda h: (0, 0)),
        ],
        out_specs=pl.BlockSpec((S, D), lambda h: (0, 0)),
        out_shape=jax.ShapeDtypeStruct((S, D), jnp.float32),
    )(att.reshape(H, S, 2 * DH), WoT, bo2)

    return out.reshape(1, S, D)


# speedup vs baseline: 82.0383x; 1.1489x over previous
import functools

import jax
import jax.numpy as jnp
from jax import lax
from jax.experimental import pallas as pl
from jax.experimental.pallas import tpu as pltpu
from jax.experimental.pallas import tpu_sc as plsc

S = 2048
D = 768
H = 12
DH = D // H
VP = 2 * DH
NW = 32
RW = H * S // NW
NCH = RW // 128
CPH = S // 128

_DN_T = (((1,), (1,)), ((), ()))
_DN_N = (((1,), (0,)), ((), ()))


def _score_idx_body(x_ref, wq_ref, bq_ref, wk_ref, bk_ref, wv_ref, bv_ref,
                    v_ref, idx_ref, q_scr, k_scr):
    h = pl.program_id(0)

    @pl.when(h == 0)
    def _():
        xb = x_ref[...]
        qb = jax.lax.dot_general(
            xb, wq_ref[...], _DN_T, preferred_element_type=jnp.float32) + bq_ref[...]
        kb = jax.lax.dot_general(
            xb, wk_ref[...], _DN_T, preferred_element_type=jnp.float32) + bk_ref[...]
        vb = jax.lax.dot_general(
            xb, wv_ref[...], _DN_T, preferred_element_type=jnp.float32) + bv_ref[...]
        for hh in range(H):
            sl = slice(hh * DH, (hh + 1) * DH)
            q_scr[hh] = qb[:, sl]
            k_scr[hh] = kb[:, sl]
            v_ref[hh, :, 0:DH] = vb[:, sl]
            v_ref[hh, :, DH:VP] = vb[:, sl]

    s = jax.lax.dot_general(q_scr[h], k_scr[h], _DN_T,
                            preferred_element_type=jnp.float32)
    idx = jnp.argmax(s, axis=1).astype(jnp.int32) + h * S
    idx_ref[...] = idx.reshape(CPH, 128)


def _make_gather():
    mesh = plsc.VectorSubcoreMesh(core_axis_name="c", subcore_axis_name="s")

    @functools.partial(
        pl.kernel, mesh=mesh,
        out_type=jax.ShapeDtypeStruct((H * S, VP), jnp.float32),
        scratch_types=[
            pltpu.VMEM((NCH, 128), jnp.int32),
            pltpu.VMEM((RW, VP), jnp.float32),
            pltpu.SemaphoreType.DMA,
        ],
    )
    def gather(table_hbm, idx_hbm, out_hbm, idx_v, rows_v, sem):
        wid = lax.axis_index("s") * 2 + lax.axis_index("c")
        pltpu.sync_copy(idx_hbm.at[wid], idx_v)
        for j in range(NCH):
            pltpu.async_copy(table_hbm.at[idx_v.at[j]],
                             rows_v.at[pl.ds(j * 128, 128)], sem).wait()
        pltpu.sync_copy(rows_v, out_hbm.at[pl.ds(wid * RW, RW)])

    return gather


def _proj_body(att_ref, wot_ref, bo_ref, out_ref):
    h = pl.program_id(0)
    proj = jax.lax.dot_general(att_ref[0, :, 0:DH], wot_ref[...], _DN_N,
                               preferred_element_type=jnp.float32)

    @pl.when(h == 0)
    def _():
        out_ref[...] = proj + bo_ref[...]

    @pl.when(h != 0)
    def _():
        out_ref[...] += proj


def kernel(x, Wq, bq, Wk, bk, Wv, bv, Wo, bo):
    x2 = x.reshape(S, D)
    bq2 = bq.reshape(1, D)
    bk2 = bk.reshape(1, D)
    bv2 = bv.reshape(1, D)
    bo2 = bo.reshape(1, D)
    WoT = Wo.T

    w_spec = pl.BlockSpec((D, D), lambda h: (0, 0))
    b_spec = pl.BlockSpec((1, D), lambda h: (0, 0))
    v, idx = pl.pallas_call(
        _score_idx_body,
        grid=(H,),
        in_specs=[pl.BlockSpec((S, D), lambda h: (0, 0)),
                  w_spec, b_spec, w_spec, b_spec, w_spec, b_spec],
        out_specs=[pl.BlockSpec((H, S, VP), lambda h: (0, 0, 0)),
                   pl.BlockSpec((CPH, 128), lambda h: (h, 0))],
        out_shape=[jax.ShapeDtypeStruct((H, S, VP), jnp.float32),
                   jax.ShapeDtypeStruct((H * CPH, 128), jnp.int32)],
        scratch_shapes=[pltpu.VMEM((H, S, DH), jnp.float32),
                        pltpu.VMEM((H, S, DH), jnp.float32)],
    )(x2, Wq, bq2, Wk, bk2, Wv, bv2)

    att = _make_gather()(v.reshape(H * S, VP),
                         idx.reshape(NW, NCH, 128))

    out = pl.pallas_call(
        _proj_body,
        grid=(H,),
        in_specs=[
            pl.BlockSpec((1, S, VP), lambda h: (h, 0, 0)),
            pl.BlockSpec((DH, D), lambda h: (h, 0)),
            pl.BlockSpec((1, D), lambda h: (0, 0)),
        ],
        out_specs=pl.BlockSpec((S, D), lambda h: (0, 0)),
        out_shape=jax.ShapeDtypeStruct((S, D), jnp.float32),
    )(att.reshape(H, S, VP), WoT, bo2)

    return out.reshape(1, S, D)


# speedup vs baseline: 84.2028x; 1.0264x over previous
import functools

import jax
import jax.numpy as jnp
from jax import lax
from jax.experimental import pallas as pl
from jax.experimental.pallas import tpu as pltpu
from jax.experimental.pallas import tpu_sc as plsc

S = 2048
D = 768
H = 12
DH = D // H
VP = 2 * DH
NW = 32
RW = H * S // NW
NCH = RW // 128
CPH = S // 128

_DN_T = (((1,), (1,)), ((), ()))
_DN_N = (((1,), (0,)), ((), ()))


def _score_idx_body(x_ref, wq_ref, bq_ref, wk_ref, bk_ref, wv_ref, bv_ref,
                    v_ref, idx_ref, q_scr, k_scr):
    h = pl.program_id(0)

    @pl.when(h == 0)
    def _():
        xb = x_ref[...]
        qb = jax.lax.dot_general(
            xb, wq_ref[...], _DN_T, preferred_element_type=jnp.float32) + bq_ref[...]
        kb = jax.lax.dot_general(
            xb, wk_ref[...], _DN_T, preferred_element_type=jnp.float32) + bk_ref[...]
        vb = jax.lax.dot_general(
            xb, wv_ref[...], _DN_T, preferred_element_type=jnp.float32) + bv_ref[...]
        for hh in range(H):
            sl = slice(hh * DH, (hh + 1) * DH)
            q_scr[hh] = qb[:, sl]
            k_scr[hh] = kb[:, sl]
            v_ref[hh, :, 0:DH] = vb[:, sl]
            v_ref[hh, :, DH:VP] = vb[:, sl]

    s = jax.lax.dot_general(q_scr[h], k_scr[h], _DN_T,
                            preferred_element_type=jnp.float32)
    idx = jnp.argmax(s, axis=1).astype(jnp.int32) + h * S
    idx_ref[...] = idx.reshape(CPH, 128)


def _make_gather():
    mesh = plsc.VectorSubcoreMesh(core_axis_name="c", subcore_axis_name="s")

    @functools.partial(
        pl.kernel, mesh=mesh,
        out_type=jax.ShapeDtypeStruct((H * S, VP), jnp.float32),
        scratch_types=[
            pltpu.VMEM((NCH, 128), jnp.int32),
            pltpu.VMEM((RW, VP), jnp.float32),
            pltpu.SemaphoreType.DMA,
        ],
    )
    def gather(table_hbm, idx_hbm, out_hbm, idx_v, rows_v, sem):
        wid = lax.axis_index("s") * 2 + lax.axis_index("c")
        pltpu.sync_copy(idx_hbm.at[wid], idx_v)
        copies = [pltpu.async_copy(table_hbm.at[idx_v.at[j]],
                                   rows_v.at[pl.ds(j * 128, 128)], sem)
                  for j in range(NCH)]
        for c in copies:
            c.wait()
        pltpu.sync_copy(rows_v, out_hbm.at[pl.ds(wid * RW, RW)])

    return gather


def _proj_body(att_ref, wot_ref, bo_ref, out_ref):
    h = pl.program_id(0)
    proj = jax.lax.dot_general(att_ref[0, :, 0:DH], wot_ref[...], _DN_N,
                               preferred_element_type=jnp.float32)

    @pl.when(h == 0)
    def _():
        out_ref[...] = proj + bo_ref[...]

    @pl.when(h != 0)
    def _():
        out_ref[...] += proj


def kernel(x, Wq, bq, Wk, bk, Wv, bv, Wo, bo):
    x2 = x.reshape(S, D)
    bq2 = bq.reshape(1, D)
    bk2 = bk.reshape(1, D)
    bv2 = bv.reshape(1, D)
    bo2 = bo.reshape(1, D)
    WoT = Wo.T

    w_spec = pl.BlockSpec((D, D), lambda h: (0, 0))
    b_spec = pl.BlockSpec((1, D), lambda h: (0, 0))
    v, idx = pl.pallas_call(
        _score_idx_body,
        grid=(H,),
        in_specs=[pl.BlockSpec((S, D), lambda h: (0, 0)),
                  w_spec, b_spec, w_spec, b_spec, w_spec, b_spec],
        out_specs=[pl.BlockSpec((H, S, VP), lambda h: (0, 0, 0)),
                   pl.BlockSpec((CPH, 128), lambda h: (h, 0))],
        out_shape=[jax.ShapeDtypeStruct((H, S, VP), jnp.float32),
                   jax.ShapeDtypeStruct((H * CPH, 128), jnp.int32)],
        scratch_shapes=[pltpu.VMEM((H, S, DH), jnp.float32),
                        pltpu.VMEM((H, S, DH), jnp.float32)],
    )(x2, Wq, bq2, Wk, bk2, Wv, bv2)

    att = _make_gather()(v.reshape(H * S, VP),
                         idx.reshape(NW, NCH, 128))

    out = pl.pallas_call(
        _proj_body,
        grid=(H,),
        in_specs=[
            pl.BlockSpec((1, S, VP), lambda h: (h, 0, 0)),
            pl.BlockSpec((DH, D), lambda h: (h, 0)),
            pl.BlockSpec((1, D), lambda h: (0, 0)),
        ],
        out_specs=pl.BlockSpec((S, D), lambda h: (0, 0)),
        out_shape=jax.ShapeDtypeStruct((S, D), jnp.float32),
    )(att.reshape(H, S, VP), WoT, bo2)

    return out.reshape(1, S, D)
